# baseline (speedup 1.0000x reference)
import functools

import jax
import jax.numpy as jnp
from jax import lax
from jax.experimental import pallas as pl
from jax.experimental.pallas import tpu as pltpu
from jax.experimental.pallas import tpu_sc as plsc

VOCAB = 100001
EMBED_DIM = 32
BATCH = 16384
H1 = 64
H2 = 32

_INFO = plsc.get_sparse_core_info()
_NC, _NS = _INFO.num_cores, _INFO.num_subcores
_NW = _NC * _NS
_B_PER_W = BATCH // _NW


def _gather_body(table_hbm, idx_hbm, out_hbm, idx_v, rows_v, sem):
    wid = lax.axis_index("s") * _NC + lax.axis_index("c")
    base = wid * _B_PER_W
    pltpu.sync_copy(idx_hbm.at[pl.ds(base, _B_PER_W)], idx_v)
    pltpu.async_copy(table_hbm.at[idx_v], rows_v, sem).wait()
    pltpu.sync_copy(rows_v, out_hbm.at[pl.ds(base, _B_PER_W)])


_sc_gather = pl.kernel(
    _gather_body,
    out_type=jax.ShapeDtypeStruct((BATCH, EMBED_DIM), jnp.float32),
    mesh=plsc.VectorSubcoreMesh(core_axis_name="c", subcore_axis_name="s"),
    scratch_types=[
        pltpu.VMEM((_B_PER_W,), jnp.int32),
        pltpu.VMEM((_B_PER_W, EMBED_DIM), jnp.float32),
        pltpu.SemaphoreType.DMA,
    ],
    compiler_params=pltpu.CompilerParams(use_tc_tiling_on_sc=False),
)


_MLP_BLOCK = 2048


def _mlp_body(x_ref, w1_ref, b1_ref, w2_ref, b2_ref, o_ref):
    x = x_ref[...]
    h = jnp.maximum(
        jax.lax.dot_general(x, w1_ref[...], (((1,), (0,)), ((), ())),
                            preferred_element_type=jnp.float32) + b1_ref[...],
        0.0,
    )
    o_ref[...] = (
        jax.lax.dot_general(h, w2_ref[...], (((1,), (0,)), ((), ())),
                            preferred_element_type=jnp.float32) + b2_ref[...]
    )


def _tc_mlp(x, w1, b1, w2, b2):
    grid = (BATCH // _MLP_BLOCK,)
    return pl.pallas_call(
        _mlp_body,
        grid=grid,
        in_specs=[
            pl.BlockSpec((_MLP_BLOCK, EMBED_DIM), lambda i: (i, 0)),
            pl.BlockSpec((EMBED_DIM, H1), lambda i: (0, 0)),
            pl.BlockSpec((H1,), lambda i: (0,)),
            pl.BlockSpec((H1, H2), lambda i: (0, 0)),
            pl.BlockSpec((H2,), lambda i: (0,)),
        ],
        out_specs=pl.BlockSpec((_MLP_BLOCK, H2), lambda i: (i, 0)),
        out_shape=jax.ShapeDtypeStruct((BATCH, H2), jnp.float32),
    )(x, w1, b1, w2, b2)


@jax.jit
def kernel(user_id, emb_table, W1, b1, W2, b2):
    gathered = _sc_gather(emb_table, user_id)
    return _tc_mlp(gathered, W1, b1, W2, b2)


# speedup vs baseline: 1.3172x; 1.3172x over previous
import functools

import jax
import jax.numpy as jnp
from jax import lax
from jax.experimental import pallas as pl
from jax.experimental.pallas import tpu as pltpu
from jax.experimental.pallas import tpu_sc as plsc

VOCAB = 100001
EMBED_DIM = 32
BATCH = 16384
H1 = 64
H2 = 32

_INFO = plsc.get_sparse_core_info()
_NC, _NS = _INFO.num_cores, _INFO.num_subcores
_NW = _NC * _NS
_B_PER_W = BATCH // _NW


def _gather_body(table_hbm, idx_hbm, out_hbm, idx_v, rows_v, sem):
    wid = lax.axis_index("s") * _NC + lax.axis_index("c")
    base = wid * _B_PER_W
    pltpu.sync_copy(idx_hbm.at[pl.ds(base, _B_PER_W)], idx_v)

    def fire(g, _):
        vec = idx_v[pl.ds(g * 16, 16)]
        for j in range(16):
            r = vec[j]
            pltpu.async_copy(
                table_hbm.at[pl.ds(r, 1), :],
                rows_v.at[pl.ds(g * 16 + j, 1), :],
                sem,
            )
        return _

    lax.fori_loop(0, _B_PER_W // 16, fire, 0)

    def drain(i, _):
        pltpu.make_async_copy(
            table_hbm.at[pl.ds(0, 1), :], rows_v.at[pl.ds(i, 1), :], sem
        ).wait()
        return _

    lax.fori_loop(0, _B_PER_W, drain, 0)
    pltpu.sync_copy(rows_v, out_hbm.at[pl.ds(base, _B_PER_W)])


_sc_gather = pl.kernel(
    _gather_body,
    out_type=jax.ShapeDtypeStruct((BATCH, EMBED_DIM), jnp.float32),
    mesh=plsc.VectorSubcoreMesh(core_axis_name="c", subcore_axis_name="s"),
    scratch_types=[
        pltpu.VMEM((_B_PER_W,), jnp.int32),
        pltpu.VMEM((_B_PER_W, EMBED_DIM), jnp.float32),
        pltpu.SemaphoreType.DMA,
    ],
)


_MLP_BLOCK = 2048


def _mlp_body(x_ref, w1_ref, b1_ref, w2_ref, b2_ref, o_ref):
    x = x_ref[...]
    h = jnp.maximum(
        jax.lax.dot_general(x, w1_ref[...], (((1,), (0,)), ((), ())),
                            preferred_element_type=jnp.float32) + b1_ref[...],
        0.0,
    )
    o_ref[...] = (
        jax.lax.dot_general(h, w2_ref[...], (((1,), (0,)), ((), ())),
                            preferred_element_type=jnp.float32) + b2_ref[...]
    )


def _tc_mlp(x, w1, b1, w2, b2):
    grid = (BATCH // _MLP_BLOCK,)
    return pl.pallas_call(
        _mlp_body,
        grid=grid,
        in_specs=[
            pl.BlockSpec((_MLP_BLOCK, EMBED_DIM), lambda i: (i, 0)),
            pl.BlockSpec((EMBED_DIM, H1), lambda i: (0, 0)),
            pl.BlockSpec((H1,), lambda i: (0,)),
            pl.BlockSpec((H1, H2), lambda i: (0, 0)),
            pl.BlockSpec((H2,), lambda i: (0,)),
        ],
        out_specs=pl.BlockSpec((_MLP_BLOCK, H2), lambda i: (i, 0)),
        out_shape=jax.ShapeDtypeStruct((BATCH, H2), jnp.float32),
    )(x, w1, b1, w2, b2)


@jax.jit
def kernel(user_id, emb_table, W1, b1, W2, b2):
    gathered = _sc_gather(emb_table, user_id)
    return _tc_mlp(gathered, W1, b1, W2, b2)


# speedup vs baseline: 2.2172x; 1.6832x over previous
import functools

import jax
import jax.numpy as jnp
from jax import lax
from jax.experimental import pallas as pl
from jax.experimental.pallas import tpu as pltpu
from jax.experimental.pallas import tpu_sc as plsc

VOCAB = 100001
EMBED_DIM = 32
BATCH = 16384
H1 = 64
H2 = 32

_INFO = plsc.get_sparse_core_info()
_NC, _NS = _INFO.num_cores, _INFO.num_subcores
_NW = _NC * _NS
_CHUNK = 2048


def _gatherT_body(tableT_hbm, idx_hbm, outT_hbm, row_v, idx_v, out_v, sem):
    c = lax.axis_index("s") * _NC + lax.axis_index("c")
    pltpu.async_copy(tableT_hbm.at[c], row_v, sem)
    pltpu.sync_copy(idx_hbm, idx_v)
    pltpu.make_async_copy(tableT_hbm.at[c], row_v, sem).wait()

    def step(t, _):
        base = t * _CHUNK
        for g in range(_CHUNK // 16):
            ids = idx_v[pl.ds(base + g * 16, 16)]
            vals = plsc.load_gather(row_v, [ids])
            out_v[pl.ds(g * 16, 16)] = vals
        pltpu.sync_copy(out_v, outT_hbm.at[c, pl.ds(base, _CHUNK)])
        return _

    lax.fori_loop(0, BATCH // _CHUNK, step, 0)


_sc_gatherT = pl.kernel(
    _gatherT_body,
    out_type=jax.ShapeDtypeStruct((EMBED_DIM, BATCH), jnp.float32),
    mesh=plsc.VectorSubcoreMesh(core_axis_name="c", subcore_axis_name="s"),
    scratch_types=[
        pltpu.VMEM((VOCAB,), jnp.float32),
        pltpu.VMEM((BATCH,), jnp.int32),
        pltpu.VMEM((_CHUNK,), jnp.float32),
        pltpu.SemaphoreType.DMA,
    ],
    compiler_params=pltpu.CompilerParams(needs_layout_passes=False),
)


_MLP_BLOCK = 4096


def _mlpT_body(x_ref, w1t_ref, b1_ref, w2t_ref, b2_ref, o_ref):
    h = jnp.maximum(
        jax.lax.dot_general(w1t_ref[...], x_ref[...], (((1,), (0,)), ((), ())),
                            preferred_element_type=jnp.float32)
        + b1_ref[...][:, None],
        0.0,
    )
    o_ref[...] = (
        jax.lax.dot_general(w2t_ref[...], h, (((1,), (0,)), ((), ())),
                            preferred_element_type=jnp.float32)
        + b2_ref[...][:, None]
    )


def _tc_mlpT(xT, w1t, b1, w2t, b2):
    grid = (BATCH // _MLP_BLOCK,)
    return pl.pallas_call(
        _mlpT_body,
        grid=grid,
        in_specs=[
            pl.BlockSpec((EMBED_DIM, _MLP_BLOCK), lambda i: (0, i)),
            pl.BlockSpec((H1, EMBED_DIM), lambda i: (0, 0)),
            pl.BlockSpec((H1,), lambda i: (0,)),
            pl.BlockSpec((H2, H1), lambda i: (0, 0)),
            pl.BlockSpec((H2,), lambda i: (0,)),
        ],
        out_specs=pl.BlockSpec((H2, _MLP_BLOCK), lambda i: (0, i)),
        out_shape=jax.ShapeDtypeStruct((H2, BATCH), jnp.float32),
    )(xT, w1t, b1, w2t, b2)


@jax.jit
def kernel(user_id, emb_table, W1, b1, W2, b2):
    gatheredT = _sc_gatherT(emb_table.T, user_id)
    outT = _tc_mlpT(gatheredT, W1.T, b1, W2.T, b2)
    return outT.T


# speedup vs baseline: 2.8265x; 1.2748x over previous
import functools

import jax
import jax.numpy as jnp
from jax import lax
from jax.experimental import pallas as pl
from jax.experimental.pallas import tpu as pltpu
from jax.experimental.pallas import tpu_sc as plsc

VOCAB = 100001
EMBED_DIM = 32
BATCH = 16384
H1 = 64
H2 = 32

_INFO = plsc.get_sparse_core_info()
_NC, _NS = _INFO.num_cores, _INFO.num_subcores
_NW = _NC * _NS
_CHUNK = 2048


def _gatherT_body(
    tableT_hbm, idx_hbm, outT_hbm, row_v, idx_v, out0_v, out1_v, sem, osem
):
    c = lax.axis_index("s") * _NC + lax.axis_index("c")
    pltpu.async_copy(tableT_hbm.at[c], row_v, sem)
    pltpu.sync_copy(idx_hbm, idx_v)
    pltpu.make_async_copy(tableT_hbm.at[c], row_v, sem).wait()

    def step(t, _):
        for half, out_v in ((0, out0_v), (1, out1_v)):
            base = (2 * t + half) * _CHUNK

            @plsc.parallel_loop(0, _CHUNK // 16, unroll=8)
            def gather_grp(g):
                ids = idx_v[pl.ds(base + g * 16, 16)]
                vals = plsc.load_gather(row_v, [ids])
                out_v[pl.ds(g * 16, 16)] = vals

            @pl.when(t >= 1)
            def _drain():
                pltpu.make_async_copy(
                    out_v, outT_hbm.at[c, pl.ds(base, _CHUNK)], osem
                ).wait()

            pltpu.async_copy(out_v, outT_hbm.at[c, pl.ds(base, _CHUNK)], osem)
        return _

    lax.fori_loop(0, BATCH // (2 * _CHUNK), step, 0)
    pltpu.make_async_copy(out0_v, outT_hbm.at[c, pl.ds(0, _CHUNK)], osem).wait()
    pltpu.make_async_copy(out1_v, outT_hbm.at[c, pl.ds(0, _CHUNK)], osem).wait()


_sc_gatherT = pl.kernel(
    _gatherT_body,
    out_type=jax.ShapeDtypeStruct((EMBED_DIM, BATCH), jnp.float32),
    mesh=plsc.VectorSubcoreMesh(core_axis_name="c", subcore_axis_name="s"),
    scratch_types=[
        pltpu.VMEM((VOCAB,), jnp.float32),
        pltpu.VMEM((BATCH,), jnp.int32),
        pltpu.VMEM((_CHUNK,), jnp.float32),
        pltpu.VMEM((_CHUNK,), jnp.float32),
        pltpu.SemaphoreType.DMA,
        pltpu.SemaphoreType.DMA,
    ],
    compiler_params=pltpu.CompilerParams(needs_layout_passes=False),
)


_MLP_BLOCK = 4096


def _mlpT_body(x_ref, w1t_ref, b1_ref, w2t_ref, b2_ref, o_ref):
    h = jnp.maximum(
        jax.lax.dot_general(w1t_ref[...], x_ref[...], (((1,), (0,)), ((), ())),
                            preferred_element_type=jnp.float32)
        + b1_ref[...][:, None],
        0.0,
    )
    o_ref[...] = (
        jax.lax.dot_general(w2t_ref[...], h, (((1,), (0,)), ((), ())),
                            preferred_element_type=jnp.float32)
        + b2_ref[...][:, None]
    )


def _tc_mlpT(xT, w1t, b1, w2t, b2):
    grid = (BATCH // _MLP_BLOCK,)
    return pl.pallas_call(
        _mlpT_body,
        grid=grid,
        in_specs=[
            pl.BlockSpec((EMBED_DIM, _MLP_BLOCK), lambda i: (0, i)),
            pl.BlockSpec((H1, EMBED_DIM), lambda i: (0, 0)),
            pl.BlockSpec((H1,), lambda i: (0,)),
            pl.BlockSpec((H2, H1), lambda i: (0, 0)),
            pl.BlockSpec((H2,), lambda i: (0,)),
        ],
        out_specs=pl.BlockSpec((H2, _MLP_BLOCK), lambda i: (0, i)),
        out_shape=jax.ShapeDtypeStruct((H2, BATCH), jnp.float32),
    )(xT, w1t, b1, w2t, b2)


@jax.jit
def kernel(user_id, emb_table, W1, b1, W2, b2):
    gatheredT = _sc_gatherT(emb_table.T, user_id)
    outT = _tc_mlpT(gatheredT, W1.T, b1, W2.T, b2)
    return outT.T


# speedup vs baseline: 2.9043x; 1.0275x over previous
import functools

import jax
import jax.numpy as jnp
from jax import lax
from jax.experimental import pallas as pl
from jax.experimental.pallas import tpu as pltpu
from jax.experimental.pallas import tpu_sc as plsc

VOCAB = 100001
EMBED_DIM = 32
BATCH = 16384
H1 = 64
H2 = 32

_INFO = plsc.get_sparse_core_info()
_NC, _NS = _INFO.num_cores, _INFO.num_subcores
_NW = _NC * _NS
_CHUNK = 2048


def _gatherT_body(
    tableT_hbm, idx_hbm, outT_hbm, row_v, idx_v, out0_v, out1_v, sem, osem
):
    c = lax.axis_index("s") * _NC + lax.axis_index("c")
    pltpu.async_copy(tableT_hbm.at[c], row_v, sem)
    pltpu.sync_copy(idx_hbm, idx_v)
    pltpu.make_async_copy(tableT_hbm.at[c], row_v, sem).wait()

    def step(t, _):
        for half, out_v in ((0, out0_v), (1, out1_v)):
            base = (2 * t + half) * _CHUNK

            @plsc.parallel_loop(0, _CHUNK // 16, unroll=4)
            def gather_grp(g):
                ids = idx_v[pl.ds(base + g * 16, 16)]
                vals = plsc.load_gather(row_v, [ids])
                out_v[pl.ds(g * 16, 16)] = vals

            @pl.when(t >= 1)
            def _drain():
                pltpu.make_async_copy(
                    out_v, outT_hbm.at[c, pl.ds(base, _CHUNK)], osem
                ).wait()

            pltpu.async_copy(out_v, outT_hbm.at[c, pl.ds(base, _CHUNK)], osem)
        return _

    lax.fori_loop(0, BATCH // (2 * _CHUNK), step, 0)
    pltpu.make_async_copy(out0_v, outT_hbm.at[c, pl.ds(0, _CHUNK)], osem).wait()
    pltpu.make_async_copy(out1_v, outT_hbm.at[c, pl.ds(0, _CHUNK)], osem).wait()


_sc_gatherT = pl.kernel(
    _gatherT_body,
    out_type=jax.ShapeDtypeStruct((EMBED_DIM, BATCH), jnp.float32),
    mesh=plsc.VectorSubcoreMesh(core_axis_name="c", subcore_axis_name="s"),
    scratch_types=[
        pltpu.VMEM((VOCAB,), jnp.float32),
        pltpu.VMEM((BATCH,), jnp.int32),
        pltpu.VMEM((_CHUNK,), jnp.float32),
        pltpu.VMEM((_CHUNK,), jnp.float32),
        pltpu.SemaphoreType.DMA,
        pltpu.SemaphoreType.DMA,
    ],
    compiler_params=pltpu.CompilerParams(needs_layout_passes=False),
)


_MLP_BLOCK = 8192


def _mlpT_body(x_ref, w1t_ref, b1_ref, w2t_ref, b2_ref, o_ref):
    h = jnp.maximum(
        jax.lax.dot_general(w1t_ref[...], x_ref[...], (((1,), (0,)), ((), ())),
                            preferred_element_type=jnp.float32)
        + b1_ref[...][:, None],
        0.0,
    )
    o_ref[...] = (
        jax.lax.dot_general(w2t_ref[...], h, (((1,), (0,)), ((), ())),
                            preferred_element_type=jnp.float32)
        + b2_ref[...][:, None]
    )


def _tc_mlpT(xT, w1t, b1, w2t, b2):
    grid = (BATCH // _MLP_BLOCK,)
    return pl.pallas_call(
        _mlpT_body,
        grid=grid,
        in_specs=[
            pl.BlockSpec((EMBED_DIM, _MLP_BLOCK), lambda i: (0, i)),
            pl.BlockSpec((H1, EMBED_DIM), lambda i: (0, 0)),
            pl.BlockSpec((H1,), lambda i: (0,)),
            pl.BlockSpec((H2, H1), lambda i: (0, 0)),
            pl.BlockSpec((H2,), lambda i: (0,)),
        ],
        out_specs=pl.BlockSpec((H2, _MLP_BLOCK), lambda i: (0, i)),
        out_shape=jax.ShapeDtypeStruct((H2, BATCH), jnp.float32),
    )(xT, w1t, b1, w2t, b2)


@jax.jit
def kernel(user_id, emb_table, W1, b1, W2, b2):
    gatheredT = _sc_gatherT(emb_table.T, user_id)
    outT = _tc_mlpT(gatheredT, W1.T, b1, W2.T, b2)
    return outT.T


# speedup vs baseline: 2.9324x; 1.0097x over previous
import functools

import jax
import jax.numpy as jnp
from jax import lax
from jax.experimental import pallas as pl
from jax.experimental.pallas import tpu as pltpu
from jax.experimental.pallas import tpu_sc as plsc

VOCAB = 100001
EMBED_DIM = 32
BATCH = 16384
H1 = 64
H2 = 32

_INFO = plsc.get_sparse_core_info()
_NC, _NS = _INFO.num_cores, _INFO.num_subcores
_NW = _NC * _NS
_CHUNK = 2048


def _gatherT_body(
    tableT_hbm, idx_hbm, outT_hbm, row_v, idx_v, out0_v, out1_v, sem, osem
):
    c = lax.axis_index("s") * _NC + lax.axis_index("c")
    pltpu.async_copy(tableT_hbm.at[c], row_v, sem)
    pltpu.sync_copy(idx_hbm, idx_v)
    pltpu.make_async_copy(tableT_hbm.at[c], row_v, sem).wait()

    def step(t, _):
        for half, out_v in ((0, out0_v), (1, out1_v)):
            base = (2 * t + half) * _CHUNK

            @plsc.parallel_loop(0, _CHUNK // 16, unroll=8)
            def gather_grp(g):
                ids = idx_v[pl.ds(base + g * 16, 16)]
                vals = plsc.load_gather(row_v, [ids])
                out_v[pl.ds(g * 16, 16)] = vals

            @pl.when(t >= 1)
            def _drain():
                pltpu.make_async_copy(
                    out_v, outT_hbm.at[c, pl.ds(base, _CHUNK)], osem
                ).wait()

            pltpu.async_copy(out_v, outT_hbm.at[c, pl.ds(base, _CHUNK)], osem)
        return _

    lax.fori_loop(0, BATCH // (2 * _CHUNK), step, 0)
    pltpu.make_async_copy(out0_v, outT_hbm.at[c, pl.ds(0, _CHUNK)], osem).wait()
    pltpu.make_async_copy(out1_v, outT_hbm.at[c, pl.ds(0, _CHUNK)], osem).wait()


_sc_gatherT = pl.kernel(
    _gatherT_body,
    out_type=jax.ShapeDtypeStruct((EMBED_DIM, BATCH), jnp.float32),
    mesh=plsc.VectorSubcoreMesh(core_axis_name="c", subcore_axis_name="s"),
    scratch_types=[
        pltpu.VMEM((VOCAB,), jnp.float32),
        pltpu.VMEM((BATCH,), jnp.int32),
        pltpu.VMEM((_CHUNK,), jnp.float32),
        pltpu.VMEM((_CHUNK,), jnp.float32),
        pltpu.SemaphoreType.DMA,
        pltpu.SemaphoreType.DMA,
    ],
    compiler_params=pltpu.CompilerParams(needs_layout_passes=False),
)


_MLP_BLOCK = 8192


def _mlpT_body(x_ref, w1t_ref, b1_ref, w2t_ref, b2_ref, o_ref):
    h = jnp.maximum(
        jax.lax.dot_general(w1t_ref[...], x_ref[...], (((1,), (0,)), ((), ())),
                            preferred_element_type=jnp.float32)
        + b1_ref[...][:, None],
        0.0,
    )
    o_ref[...] = (
        jax.lax.dot_general(w2t_ref[...], h, (((1,), (0,)), ((), ())),
                            preferred_element_type=jnp.float32)
        + b2_ref[...][:, None]
    )


def _tc_mlpT(xT, w1t, b1, w2t, b2):
    grid = (BATCH // _MLP_BLOCK,)
    return pl.pallas_call(
        _mlpT_body,
        grid=grid,
        in_specs=[
            pl.BlockSpec((EMBED_DIM, _MLP_BLOCK), lambda i: (0, i)),
            pl.BlockSpec((H1, EMBED_DIM), lambda i: (0, 0)),
            pl.BlockSpec((H1,), lambda i: (0,)),
            pl.BlockSpec((H2, H1), lambda i: (0, 0)),
            pl.BlockSpec((H2,), lambda i: (0,)),
        ],
        out_specs=pl.BlockSpec((H2, _MLP_BLOCK), lambda i: (0, i)),
        out_shape=jax.ShapeDtypeStruct((H2, BATCH), jnp.float32),
    )(xT, w1t, b1, w2t, b2)


@jax.jit
def kernel(user_id, emb_table, W1, b1, W2, b2):
    gatheredT = _sc_gatherT(emb_table.T, user_id)
    outT = _tc_mlpT(gatheredT, W1.T, b1, W2.T, b2)
    return outT.T
